# baseline (speedup 1.0000x reference)
import functools

import jax
import jax.numpy as jnp
from jax import lax
from jax.experimental import pallas as pl
from jax.experimental.pallas import tpu as pltpu
from jax.experimental.pallas import tpu_sc as plsc

NC = 2
NS = 16
NW = NC * NS
G = 80

def _sc_mesh():
    return plsc.VectorSubcoreMesh(core_axis_name="c", subcore_axis_name="s")


def _sc_gather_sum(ps, pr, senders, receivers):
    E = senders.shape[0]
    Hd = ps.shape[1]
    assert E % (NW * G) == 0
    ew = E // NW
    ng = ew // G
    assert ng >= 2 and G % 16 == 0
    npairs = ng // 2

    @functools.partial(
        pl.kernel,
        mesh=_sc_mesh(),
        out_type=jax.ShapeDtypeStruct((E, Hd), jnp.float32),
        scratch_types=[
            pltpu.VMEM_SHARED((2 * NS * G, Hd), jnp.float32),
            pltpu.VMEM((ew,), jnp.int32),
            pltpu.VMEM((ew,), jnp.int32),
            pltpu.VMEM((G,), jnp.int32),
            pltpu.VMEM((G,), jnp.int32),
            pltpu.VMEM((G, Hd), jnp.float32),
            pltpu.VMEM((G, Hd), jnp.float32),
            pltpu.VMEM((G, Hd), jnp.float32),
            pltpu.VMEM((G, Hd), jnp.float32),
            pltpu.SemaphoreType.DMA,
            pltpu.SemaphoreType.DMA,
            pltpu.SemaphoreType.DMA,
            pltpu.SemaphoreType.DMA,
        ],
    )
    def k(ps_hbm, pr_hbm, s_hbm, r_hbm, gs_hbm, sh, si, ri, idv_a, idv_b,
          s_a, s_b, r_a, r_b, sem_sa, sem_ra, sem_sb, sem_rb):
        cid = lax.axis_index("c")
        sid = lax.axis_index("s")
        wid = sid * NC + cid
        base = wid * ew
        slot_a = sid * 2 * G
        slot_b = slot_a + G
        pltpu.sync_copy(s_hbm.at[pl.ds(base, ew)], si)
        pltpu.sync_copy(r_hbm.at[pl.ds(base, ew)], ri)
        for c in range(G // 16):
            chunk = lax.iota(jnp.int32, 16) + (16 * c + slot_a)
            idv_a[pl.ds(16 * c, 16)] = chunk
            idv_b[pl.ds(16 * c, 16)] = chunk + G

        def start(g, sbuf, rbuf, ss, sr):
            pltpu.async_copy(ps_hbm.at[si.at[pl.ds(g * G, G)]], sbuf, ss)
            pltpu.async_copy(pr_hbm.at[ri.at[pl.ds(g * G, G)]], rbuf, sr)

        def finish(g, slot, idv, sbuf, rbuf, ss, sr):
            pltpu.make_async_copy(ps_hbm.at[si.at[pl.ds(g * G, G)]], sbuf,
                                  ss).wait()
            pltpu.make_async_copy(pr_hbm.at[ri.at[pl.ds(g * G, G)]], rbuf,
                                  sr).wait()
            pltpu.sync_copy(sbuf, sh.at[pl.ds(slot, G)])
            pltpu.sync_copy(rbuf, sh.at[idv], add=True)
            pltpu.sync_copy(sh.at[pl.ds(slot, G)],
                            gs_hbm.at[pl.ds(base + g * G, G)])

        start(0, s_a, r_a, sem_sa, sem_ra)

        @pl.loop(0, npairs)
        def _(i):
            start(2 * i + 1, s_b, r_b, sem_sb, sem_rb)
            finish(2 * i, slot_a, idv_a, s_a, r_a, sem_sa, sem_ra)

            @pl.when(2 * i + 2 < ng)
            def _():
                start(2 * i + 2, s_a, r_a, sem_sa, sem_ra)

            finish(2 * i + 1, slot_b, idv_b, s_b, r_b, sem_sb, sem_rb)

        if ng % 2 == 1:
            finish(ng - 1, slot_a, idv_a, s_a, r_a, sem_sa, sem_ra)

    return k(ps, pr, senders, receivers)


def _sc_scatter_add(e_new, receivers, init):
    E, Hd = e_new.shape
    N = init.shape[0] // NC
    assert E % (NC * NS * G) == 0
    ec = E // NC
    ess = ec // NS
    ng = ess // G
    zg = -(-N // G)
    assert N % G == 0
    zrounds = -(-zg // NS)

    @functools.partial(
        pl.kernel,
        mesh=_sc_mesh(),
        out_type=jax.ShapeDtypeStruct((NC * N, Hd), jnp.float32),
        scratch_types=[
            pltpu.VMEM_SHARED((N, Hd), jnp.float32),
            pltpu.VMEM((G,), jnp.int32),
            pltpu.VMEM((G, Hd), jnp.float32),
        ],
    )
    def k(e_hbm, r_hbm, z_hbm, out_hbm, acc_sh, idx_v, rows_v):
        cid = lax.axis_index("c")
        sid = lax.axis_index("s")

        @pl.loop(0, zrounds)
        def _(j):
            gi = j * NS + sid

            @pl.when(gi < zg)
            def _():
                pltpu.sync_copy(z_hbm.at[pl.ds(cid * N + gi * G, G)],
                                acc_sh.at[pl.ds(gi * G, G)])

        plsc.subcore_barrier()

        @pl.loop(0, ng)
        def _(g):
            off = cid * ec + sid * ess + g * G
            pltpu.sync_copy(r_hbm.at[pl.ds(off, G)], idx_v)
            pltpu.sync_copy(e_hbm.at[pl.ds(off, G)], rows_v)
            pltpu.sync_copy(rows_v, acc_sh.at[idx_v], add=True)

        plsc.subcore_barrier()

        @pl.loop(0, zrounds)
        def _(j):
            gi = j * NS + sid

            @pl.when(gi < zg)
            def _():
                pltpu.sync_copy(acc_sh.at[pl.ds(gi * G, G)],
                                out_hbm.at[pl.ds(cid * N + gi * G, G)])

    return k(e_new, receivers, init)


def _ln(y, g, bl):
    mu = jnp.mean(y, axis=-1, keepdims=True)
    var = jnp.mean((y - mu) * (y - mu), axis=-1, keepdims=True)
    return (y - mu) * lax.rsqrt(var + 1e-5) * g + bl


def _dot(a, b):
    return jnp.dot(a, b, preferred_element_type=jnp.float32)


def _full_spec(shape):
    return pl.BlockSpec(shape, lambda i: tuple(0 for _ in shape))


def _mlp_ln_kernel(x_ref, w1_ref, b1_ref, w2_ref, b2_ref, g_ref, bl_ref,
                   o_ref):
    z = jnp.maximum(_dot(x_ref[...], w1_ref[...]) + b1_ref[...], 0.0)
    y = _dot(z, w2_ref[...]) + b2_ref[...]
    o_ref[...] = _ln(y, g_ref[...], bl_ref[...])


def _encode(x, p, tile):
    n, d = x.shape
    assert n % tile == 0
    H = p["w1"].shape[1]
    return pl.pallas_call(
        _mlp_ln_kernel,
        grid=(n // tile,),
        in_specs=[
            pl.BlockSpec((tile, d), lambda i: (i, 0)),
            _full_spec(p["w1"].shape),
            _full_spec((1, H)),
            _full_spec(p["w2"].shape),
            _full_spec((1, p["w2"].shape[1])),
            _full_spec((1, p["w2"].shape[1])),
            _full_spec((1, p["w2"].shape[1])),
        ],
        out_specs=pl.BlockSpec((tile, p["w2"].shape[1]), lambda i: (i, 0)),
        out_shape=jax.ShapeDtypeStruct((n, p["w2"].shape[1]), jnp.float32),
    )(x, p["w1"], p["b1"].reshape(1, -1), p["w2"], p["b2"].reshape(1, -1),
      p["g"].reshape(1, -1), p["bl"].reshape(1, -1))


def _proj_kernel(h_ref, ws_ref, wr_ref, ps_ref, pr_ref):
    ps_ref[...] = _dot(h_ref[...], ws_ref[...])
    pr_ref[...] = _dot(h_ref[...], wr_ref[...])


def _project(h, ws, wr, tile):
    N, H = h.shape
    assert N % tile == 0
    row = pl.BlockSpec((tile, H), lambda i: (i, 0))
    out = jax.ShapeDtypeStruct((N, H), jnp.float32)
    return pl.pallas_call(
        _proj_kernel,
        grid=(N // tile,),
        in_specs=[row, _full_spec((H, H)), _full_spec((H, H))],
        out_specs=(row, row),
        out_shape=(out, out),
    )(h, ws, wr)


def _edge_kernel(gs_ref, e_ref, w1e_ref, b1_ref, w2_ref, b2_ref,
                 g_ref, bl_ref, enew_ref, enext_ref):
    z = (gs_ref[...] + _dot(e_ref[...], w1e_ref[...]) + b1_ref[...])
    z = jnp.maximum(z, 0.0)
    y = _dot(z, w2_ref[...]) + b2_ref[...]
    e_new = _ln(y, g_ref[...], bl_ref[...])
    enew_ref[...] = e_new
    enext_ref[...] = e_ref[...] + e_new


def _edge_mlp(gsum, e, p, tile):
    E, H = e.shape
    assert E % tile == 0
    out = jax.ShapeDtypeStruct((E, H), jnp.float32)
    row = pl.BlockSpec((tile, H), lambda i: (i, 0))
    return pl.pallas_call(
        _edge_kernel,
        grid=(E // tile,),
        in_specs=[row, row,
                  _full_spec((H, H)), _full_spec((1, H)),
                  _full_spec((H, H)), _full_spec((1, H)),
                  _full_spec((1, H)), _full_spec((1, H))],
        out_specs=(row, row),
        out_shape=(out, out),
    )(gsum, e, p["w1"][2 * H:3 * H], p["b1"].reshape(1, -1), p["w2"],
      p["b2"].reshape(1, -1), p["g"].reshape(1, -1), p["bl"].reshape(1, -1))


def _node_kernel(h_ref, a0_ref, a1_ref, w1_ref, b1_ref, w2_ref, b2_ref,
                 g_ref, bl_ref, o_ref):
    H = h_ref.shape[1]
    agg = a0_ref[...] + a1_ref[...]
    z = (_dot(h_ref[...], w1_ref[0:H])
         + _dot(agg, w1_ref[H:2 * H])
         + b1_ref[...])
    z = jnp.maximum(z, 0.0)
    y = _dot(z, w2_ref[...]) + b2_ref[...]
    o_ref[...] = h_ref[...] + _ln(y, g_ref[...], bl_ref[...])


def _node_mlp(h, partials, p, tile):
    N, H = h.shape
    assert N % tile == 0
    nb = N // tile
    row = pl.BlockSpec((tile, H), lambda i: (i, 0))
    return pl.pallas_call(
        _node_kernel,
        grid=(nb,),
        in_specs=[row,
                  pl.BlockSpec((tile, H), lambda i: (i, 0)),
                  pl.BlockSpec((tile, H), lambda i: (i + nb, 0)),
                  _full_spec((2 * H, H)), _full_spec((1, H)),
                  _full_spec((H, H)), _full_spec((1, H)),
                  _full_spec((1, H)), _full_spec((1, H))],
        out_specs=row,
        out_shape=jax.ShapeDtypeStruct((N, H), jnp.float32),
    )(h, partials, partials, p["w1"], p["b1"].reshape(1, -1), p["w2"],
      p["b2"].reshape(1, -1), p["g"].reshape(1, -1), p["bl"].reshape(1, -1))


def _dec_kernel(h_ref, w1_ref, b1_ref, w2_ref, b2_ref, o_ref):
    z = jnp.maximum(_dot(h_ref[...], w1_ref[...]) + b1_ref[...], 0.0)
    o_ref[...] = _dot(z, w2_ref[...]) + b2_ref[...]


def _decode(h, p, tile):
    N, H = h.shape
    out_d = p["w2"].shape[1]
    return pl.pallas_call(
        _dec_kernel,
        grid=(N // tile,),
        in_specs=[pl.BlockSpec((tile, H), lambda i: (i, 0)),
                  _full_spec((H, H)), _full_spec((1, H)),
                  _full_spec((H, out_d)), _full_spec((1, out_d))],
        out_specs=pl.BlockSpec((tile, out_d), lambda i: (i, 0)),
        out_shape=jax.ShapeDtypeStruct((N, out_d), jnp.float32),
    )(h, p["w1"], p["b1"].reshape(1, -1), p["w2"], p["b2"].reshape(1, -1))


def kernel(x, edge_index, edge_attr, params):
    N = x.shape[0]
    E = edge_attr.shape[0]
    H = params["enc_node"]["w2"].shape[1]
    senders = edge_index[0]
    receivers = edge_index[1]

    h = _encode(x, params["enc_node"], tile=2000)
    e = _encode(edge_attr, params["enc_edge"], tile=2560)
    zeros = jnp.zeros((NC * N, H), jnp.float32)

    NCHUNK = 4
    unit = NW * G
    groups = E // unit
    per = groups // NCHUNK
    sizes = [per * unit] * (NCHUNK - 1)
    sizes.append(E - sum(sizes))
    offs = [sum(sizes[:i]) for i in range(NCHUNK)]
    sch = [senders[o:o + n] for o, n in zip(offs, sizes)]
    rch = [receivers[o:o + n] for o, n in zip(offs, sizes)]
    ech = [e[o:o + n] for o, n in zip(offs, sizes)]

    for blk in params["blocks"]:
        w1 = blk["edge"]["w1"]
        ps, pr = _project(h, w1[0:H], w1[H:2 * H], tile=2000)
        gs = [_sc_gather_sum(ps, pr, s, r) for s, r in zip(sch, rch)]
        en = []
        for i in range(NCHUNK):
            e_new, ech[i] = _edge_mlp(gs[i], ech[i], blk["edge"], tile=2560)
            en.append(e_new)
        p = zeros
        for i in range(NCHUNK):
            p = _sc_scatter_add(en[i], rch[i], p)
        h = _node_mlp(h, p, blk["node"], tile=2000)

    return _decode(h, params["dec"], tile=2000)


# speedup vs baseline: 1.0652x; 1.0652x over previous
import functools

import jax
import jax.numpy as jnp
from jax import lax
from jax.experimental import pallas as pl
from jax.experimental.pallas import tpu as pltpu
from jax.experimental.pallas import tpu_sc as plsc

NC = 2
NS = 16
NW = NC * NS
G = 80

def _sc_mesh():
    return plsc.VectorSubcoreMesh(core_axis_name="c", subcore_axis_name="s")


def _sc_gather_sum(ps, pr, senders, receivers):
    E = senders.shape[0]
    Hd = ps.shape[1]
    assert E % (NW * G) == 0
    ew = E // NW
    ng = ew // G
    assert ng >= 2 and G % 16 == 0
    npairs = ng // 2

    @functools.partial(
        pl.kernel,
        mesh=_sc_mesh(),
        out_type=jax.ShapeDtypeStruct((E, Hd), jnp.float32),
        scratch_types=[
            pltpu.VMEM_SHARED((2 * NS * G, Hd), jnp.float32),
            pltpu.VMEM((ew,), jnp.int32),
            pltpu.VMEM((ew,), jnp.int32),
            pltpu.VMEM((G,), jnp.int32),
            pltpu.VMEM((G,), jnp.int32),
            pltpu.VMEM((G, Hd), jnp.float32),
            pltpu.VMEM((G, Hd), jnp.float32),
            pltpu.VMEM((G, Hd), jnp.float32),
            pltpu.VMEM((G, Hd), jnp.float32),
            pltpu.SemaphoreType.DMA,
            pltpu.SemaphoreType.DMA,
            pltpu.SemaphoreType.DMA,
            pltpu.SemaphoreType.DMA,
        ],
    )
    def k(ps_hbm, pr_hbm, s_hbm, r_hbm, gs_hbm, sh, si, ri, idv_a, idv_b,
          s_a, s_b, r_a, r_b, sem_sa, sem_ra, sem_sb, sem_rb):
        cid = lax.axis_index("c")
        sid = lax.axis_index("s")
        wid = sid * NC + cid
        base = wid * ew
        slot_a = sid * 2 * G
        slot_b = slot_a + G
        pltpu.sync_copy(s_hbm.at[pl.ds(base, ew)], si)
        pltpu.sync_copy(r_hbm.at[pl.ds(base, ew)], ri)
        for c in range(G // 16):
            chunk = lax.iota(jnp.int32, 16) + (16 * c + slot_a)
            idv_a[pl.ds(16 * c, 16)] = chunk
            idv_b[pl.ds(16 * c, 16)] = chunk + G

        def start(g, sbuf, rbuf, ss, sr):
            pltpu.async_copy(ps_hbm.at[si.at[pl.ds(g * G, G)]], sbuf, ss)
            pltpu.async_copy(pr_hbm.at[ri.at[pl.ds(g * G, G)]], rbuf, sr)

        def finish(g, slot, idv, sbuf, rbuf, ss, sr):
            pltpu.make_async_copy(ps_hbm.at[si.at[pl.ds(g * G, G)]], sbuf,
                                  ss).wait()
            pltpu.make_async_copy(pr_hbm.at[ri.at[pl.ds(g * G, G)]], rbuf,
                                  sr).wait()
            pltpu.sync_copy(sbuf, sh.at[pl.ds(slot, G)])
            pltpu.sync_copy(rbuf, sh.at[idv], add=True)
            pltpu.sync_copy(sh.at[pl.ds(slot, G)],
                            gs_hbm.at[pl.ds(base + g * G, G)])

        start(0, s_a, r_a, sem_sa, sem_ra)

        @pl.loop(0, npairs)
        def _(i):
            start(2 * i + 1, s_b, r_b, sem_sb, sem_rb)
            finish(2 * i, slot_a, idv_a, s_a, r_a, sem_sa, sem_ra)

            @pl.when(2 * i + 2 < ng)
            def _():
                start(2 * i + 2, s_a, r_a, sem_sa, sem_ra)

            finish(2 * i + 1, slot_b, idv_b, s_b, r_b, sem_sb, sem_rb)

        if ng % 2 == 1:
            finish(ng - 1, slot_a, idv_a, s_a, r_a, sem_sa, sem_ra)

    return k(ps, pr, senders, receivers)


def _sc_scatter_add(e_new, receivers, init):
    E, Hd = e_new.shape
    N = init.shape[0] // NC
    assert E % (NC * NS * G) == 0
    ec = E // NC
    ess = ec // NS
    ng = ess // G
    zg = -(-N // G)
    assert N % G == 0
    zrounds = -(-zg // NS)

    @functools.partial(
        pl.kernel,
        mesh=_sc_mesh(),
        out_type=jax.ShapeDtypeStruct((NC * N, Hd), jnp.float32),
        scratch_types=[
            pltpu.VMEM_SHARED((N, Hd), jnp.float32),
            pltpu.VMEM((G,), jnp.int32),
            pltpu.VMEM((G, Hd), jnp.float32),
        ],
    )
    def k(e_hbm, r_hbm, z_hbm, out_hbm, acc_sh, idx_v, rows_v):
        cid = lax.axis_index("c")
        sid = lax.axis_index("s")

        @pl.loop(0, zrounds)
        def _(j):
            gi = j * NS + sid

            @pl.when(gi < zg)
            def _():
                pltpu.sync_copy(z_hbm.at[pl.ds(cid * N + gi * G, G)],
                                acc_sh.at[pl.ds(gi * G, G)])

        plsc.subcore_barrier()

        @pl.loop(0, ng)
        def _(g):
            off = cid * ec + sid * ess + g * G
            pltpu.sync_copy(r_hbm.at[pl.ds(off, G)], idx_v)
            pltpu.sync_copy(e_hbm.at[pl.ds(off, G)], rows_v)
            pltpu.sync_copy(rows_v, acc_sh.at[idx_v], add=True)

        plsc.subcore_barrier()

        @pl.loop(0, zrounds)
        def _(j):
            gi = j * NS + sid

            @pl.when(gi < zg)
            def _():
                pltpu.sync_copy(acc_sh.at[pl.ds(gi * G, G)],
                                out_hbm.at[pl.ds(cid * N + gi * G, G)])

    return k(e_new, receivers, init)


def _ln(y, g, bl):
    mu = jnp.mean(y, axis=-1, keepdims=True)
    var = jnp.mean((y - mu) * (y - mu), axis=-1, keepdims=True)
    return (y - mu) * lax.rsqrt(var + 1e-5) * g + bl


def _dot(a, b):
    return jnp.dot(a, b, preferred_element_type=jnp.float32)


def _full_spec(shape):
    return pl.BlockSpec(shape, lambda i: tuple(0 for _ in shape))


def _mlp_ln_kernel(x_ref, w1_ref, b1_ref, w2_ref, b2_ref, g_ref, bl_ref,
                   o_ref):
    z = jnp.maximum(_dot(x_ref[...], w1_ref[...]) + b1_ref[...], 0.0)
    y = _dot(z, w2_ref[...]) + b2_ref[...]
    o_ref[...] = _ln(y, g_ref[...], bl_ref[...])


def _encode(x, p, tile):
    n, d = x.shape
    assert n % tile == 0
    H = p["w1"].shape[1]
    return pl.pallas_call(
        _mlp_ln_kernel,
        grid=(n // tile,),
        in_specs=[
            pl.BlockSpec((tile, d), lambda i: (i, 0)),
            _full_spec(p["w1"].shape),
            _full_spec((1, H)),
            _full_spec(p["w2"].shape),
            _full_spec((1, p["w2"].shape[1])),
            _full_spec((1, p["w2"].shape[1])),
            _full_spec((1, p["w2"].shape[1])),
        ],
        out_specs=pl.BlockSpec((tile, p["w2"].shape[1]), lambda i: (i, 0)),
        out_shape=jax.ShapeDtypeStruct((n, p["w2"].shape[1]), jnp.float32),
    )(x, p["w1"], p["b1"].reshape(1, -1), p["w2"], p["b2"].reshape(1, -1),
      p["g"].reshape(1, -1), p["bl"].reshape(1, -1))


def _proj_kernel(h_ref, ws_ref, wr_ref, ps_ref, pr_ref):
    ps_ref[...] = _dot(h_ref[...], ws_ref[...])
    pr_ref[...] = _dot(h_ref[...], wr_ref[...])


def _project(h, ws, wr, tile):
    N, H = h.shape
    assert N % tile == 0
    row = pl.BlockSpec((tile, H), lambda i: (i, 0))
    out = jax.ShapeDtypeStruct((N, H), jnp.float32)
    return pl.pallas_call(
        _proj_kernel,
        grid=(N // tile,),
        in_specs=[row, _full_spec((H, H)), _full_spec((H, H))],
        out_specs=(row, row),
        out_shape=(out, out),
    )(h, ws, wr)


def _edge_kernel(gs_ref, e_ref, w1e_ref, b1_ref, w2_ref, b2_ref,
                 g_ref, bl_ref, enew_ref, enext_ref):
    z = (gs_ref[...] + _dot(e_ref[...], w1e_ref[...]) + b1_ref[...])
    z = jnp.maximum(z, 0.0)
    y = _dot(z, w2_ref[...]) + b2_ref[...]
    e_new = _ln(y, g_ref[...], bl_ref[...])
    enew_ref[...] = e_new
    enext_ref[...] = e_ref[...] + e_new


def _edge_mlp(gsum, e, p, tile):
    E, H = e.shape
    assert E % tile == 0
    out = jax.ShapeDtypeStruct((E, H), jnp.float32)
    row = pl.BlockSpec((tile, H), lambda i: (i, 0))
    return pl.pallas_call(
        _edge_kernel,
        grid=(E // tile,),
        in_specs=[row, row,
                  _full_spec((H, H)), _full_spec((1, H)),
                  _full_spec((H, H)), _full_spec((1, H)),
                  _full_spec((1, H)), _full_spec((1, H))],
        out_specs=(row, row),
        out_shape=(out, out),
    )(gsum, e, p["w1"][2 * H:3 * H], p["b1"].reshape(1, -1), p["w2"],
      p["b2"].reshape(1, -1), p["g"].reshape(1, -1), p["bl"].reshape(1, -1))


def _node_kernel(h_ref, a0_ref, a1_ref, w1_ref, b1_ref, w2_ref, b2_ref,
                 g_ref, bl_ref, o_ref):
    H = h_ref.shape[1]
    agg = a0_ref[...] + a1_ref[...]
    z = (_dot(h_ref[...], w1_ref[0:H])
         + _dot(agg, w1_ref[H:2 * H])
         + b1_ref[...])
    z = jnp.maximum(z, 0.0)
    y = _dot(z, w2_ref[...]) + b2_ref[...]
    o_ref[...] = h_ref[...] + _ln(y, g_ref[...], bl_ref[...])


def _node_mlp(h, partials, p, tile):
    N, H = h.shape
    assert N % tile == 0
    nb = N // tile
    row = pl.BlockSpec((tile, H), lambda i: (i, 0))
    return pl.pallas_call(
        _node_kernel,
        grid=(nb,),
        in_specs=[row,
                  pl.BlockSpec((tile, H), lambda i: (i, 0)),
                  pl.BlockSpec((tile, H), lambda i: (i + nb, 0)),
                  _full_spec((2 * H, H)), _full_spec((1, H)),
                  _full_spec((H, H)), _full_spec((1, H)),
                  _full_spec((1, H)), _full_spec((1, H))],
        out_specs=row,
        out_shape=jax.ShapeDtypeStruct((N, H), jnp.float32),
    )(h, partials, partials, p["w1"], p["b1"].reshape(1, -1), p["w2"],
      p["b2"].reshape(1, -1), p["g"].reshape(1, -1), p["bl"].reshape(1, -1))


def _dec_kernel(h_ref, w1_ref, b1_ref, w2_ref, b2_ref, o_ref):
    z = jnp.maximum(_dot(h_ref[...], w1_ref[...]) + b1_ref[...], 0.0)
    o_ref[...] = _dot(z, w2_ref[...]) + b2_ref[...]


def _decode(h, p, tile):
    N, H = h.shape
    out_d = p["w2"].shape[1]
    return pl.pallas_call(
        _dec_kernel,
        grid=(N // tile,),
        in_specs=[pl.BlockSpec((tile, H), lambda i: (i, 0)),
                  _full_spec((H, H)), _full_spec((1, H)),
                  _full_spec((H, out_d)), _full_spec((1, out_d))],
        out_specs=pl.BlockSpec((tile, out_d), lambda i: (i, 0)),
        out_shape=jax.ShapeDtypeStruct((N, out_d), jnp.float32),
    )(h, p["w1"], p["b1"].reshape(1, -1), p["w2"], p["b2"].reshape(1, -1))


def kernel(x, edge_index, edge_attr, params):
    N = x.shape[0]
    E = edge_attr.shape[0]
    H = params["enc_node"]["w2"].shape[1]
    senders = edge_index[0]
    receivers = edge_index[1]

    h = _encode(x, params["enc_node"], tile=2000)
    e = _encode(edge_attr, params["enc_edge"], tile=2560)
    zeros = jnp.zeros((NC * N, H), jnp.float32)

    NCHUNK = 2
    unit = NW * G
    groups = E // unit
    per = groups // NCHUNK
    sizes = [per * unit] * (NCHUNK - 1)
    sizes.append(E - sum(sizes))
    offs = [sum(sizes[:i]) for i in range(NCHUNK)]
    sch = [senders[o:o + n] for o, n in zip(offs, sizes)]
    rch = [receivers[o:o + n] for o, n in zip(offs, sizes)]
    ech = [e[o:o + n] for o, n in zip(offs, sizes)]

    for blk in params["blocks"]:
        w1 = blk["edge"]["w1"]
        ps, pr = _project(h, w1[0:H], w1[H:2 * H], tile=2000)
        gs = [_sc_gather_sum(ps, pr, s, r) for s, r in zip(sch, rch)]
        en = []
        for i in range(NCHUNK):
            e_new, ech[i] = _edge_mlp(gs[i], ech[i], blk["edge"], tile=2560)
            en.append(e_new)
        p = zeros
        for i in range(NCHUNK):
            p = _sc_scatter_add(en[i], rch[i], p)
        h = _node_mlp(h, p, blk["node"], tile=2000)

    return _decode(h, params["dec"], tile=2000)


# speedup vs baseline: 1.2558x; 1.1789x over previous
import functools

import jax
import jax.numpy as jnp
from jax import lax
from jax.experimental import pallas as pl
from jax.experimental.pallas import tpu as pltpu
from jax.experimental.pallas import tpu_sc as plsc

NC = 2
NS = 16
NW = NC * NS
G = 80

def _sc_mesh():
    return plsc.VectorSubcoreMesh(core_axis_name="c", subcore_axis_name="s")


def _sc_gather_sum(ps, pr, senders, receivers):
    E = senders.shape[0]
    Hd = ps.shape[1]
    assert E % (NW * G) == 0
    ew = E // NW
    ng = ew // G
    assert ng >= 2 and G % 16 == 0
    npairs = ng // 2

    @functools.partial(
        pl.kernel,
        mesh=_sc_mesh(),
        out_type=jax.ShapeDtypeStruct((E, Hd), jnp.float32),
        scratch_types=[
            pltpu.VMEM_SHARED((2 * NS * G, Hd), jnp.float32),
            pltpu.VMEM((ew,), jnp.int32),
            pltpu.VMEM((ew,), jnp.int32),
            pltpu.VMEM((G,), jnp.int32),
            pltpu.VMEM((G,), jnp.int32),
            pltpu.VMEM((G, Hd), jnp.float32),
            pltpu.VMEM((G, Hd), jnp.float32),
            pltpu.VMEM((G, Hd), jnp.float32),
            pltpu.VMEM((G, Hd), jnp.float32),
            pltpu.SemaphoreType.DMA,
            pltpu.SemaphoreType.DMA,
            pltpu.SemaphoreType.DMA,
            pltpu.SemaphoreType.DMA,
        ],
    )
    def k(ps_hbm, pr_hbm, s_hbm, r_hbm, gs_hbm, sh, si, ri, idv_a, idv_b,
          s_a, s_b, r_a, r_b, sem_sa, sem_ra, sem_sb, sem_rb):
        cid = lax.axis_index("c")
        sid = lax.axis_index("s")
        wid = sid * NC + cid
        base = wid * ew
        slot_a = sid * 2 * G
        slot_b = slot_a + G
        pltpu.sync_copy(s_hbm.at[pl.ds(base, ew)], si)
        pltpu.sync_copy(r_hbm.at[pl.ds(base, ew)], ri)
        for c in range(G // 16):
            chunk = lax.iota(jnp.int32, 16) + (16 * c + slot_a)
            idv_a[pl.ds(16 * c, 16)] = chunk
            idv_b[pl.ds(16 * c, 16)] = chunk + G

        def start(g, sbuf, rbuf, ss, sr):
            pltpu.async_copy(ps_hbm.at[si.at[pl.ds(g * G, G)]], sbuf, ss)
            pltpu.async_copy(pr_hbm.at[ri.at[pl.ds(g * G, G)]], rbuf, sr)

        def finish(g, slot, idv, sbuf, rbuf, ss, sr):
            pltpu.make_async_copy(ps_hbm.at[si.at[pl.ds(g * G, G)]], sbuf,
                                  ss).wait()
            pltpu.make_async_copy(pr_hbm.at[ri.at[pl.ds(g * G, G)]], rbuf,
                                  sr).wait()
            pltpu.sync_copy(sbuf, sh.at[pl.ds(slot, G)])
            pltpu.sync_copy(rbuf, sh.at[idv], add=True)
            pltpu.sync_copy(sh.at[pl.ds(slot, G)],
                            gs_hbm.at[pl.ds(base + g * G, G)])

        start(0, s_a, r_a, sem_sa, sem_ra)

        @pl.loop(0, npairs)
        def _(i):
            start(2 * i + 1, s_b, r_b, sem_sb, sem_rb)
            finish(2 * i, slot_a, idv_a, s_a, r_a, sem_sa, sem_ra)

            @pl.when(2 * i + 2 < ng)
            def _():
                start(2 * i + 2, s_a, r_a, sem_sa, sem_ra)

            finish(2 * i + 1, slot_b, idv_b, s_b, r_b, sem_sb, sem_rb)

        if ng % 2 == 1:
            finish(ng - 1, slot_a, idv_a, s_a, r_a, sem_sa, sem_ra)

    return k(ps, pr, senders, receivers)


def _sc_scatter_add(e_new, receivers, init):
    E, Hd = e_new.shape
    N = init.shape[0] // NC
    assert E % (NC * NS * G) == 0
    ec = E // NC
    ess = ec // NS
    ng = ess // G
    zg = -(-N // G)
    assert N % G == 0
    zrounds = -(-zg // NS)

    assert ng >= 2
    npairs = ng // 2

    @functools.partial(
        pl.kernel,
        mesh=_sc_mesh(),
        out_type=jax.ShapeDtypeStruct((NC * N, Hd), jnp.float32),
        scratch_types=[
            pltpu.VMEM_SHARED((N, Hd), jnp.float32),
            pltpu.VMEM((G,), jnp.int32),
            pltpu.VMEM((G,), jnp.int32),
            pltpu.VMEM((G, Hd), jnp.float32),
            pltpu.VMEM((G, Hd), jnp.float32),
            pltpu.SemaphoreType.DMA,
            pltpu.SemaphoreType.DMA,
            pltpu.SemaphoreType.DMA,
            pltpu.SemaphoreType.DMA,
        ],
    )
    def k(e_hbm, r_hbm, z_hbm, out_hbm, acc_sh, idx_a, idx_b, rows_a, rows_b,
          sem_ia, sem_ra, sem_ib, sem_rb):
        cid = lax.axis_index("c")
        sid = lax.axis_index("s")

        @pl.loop(0, zrounds)
        def _(j):
            gi = j * NS + sid

            @pl.when(gi < zg)
            def _():
                pltpu.sync_copy(z_hbm.at[pl.ds(cid * N + gi * G, G)],
                                acc_sh.at[pl.ds(gi * G, G)])

        plsc.subcore_barrier()

        def start(g, ibuf, rbuf, si, sr):
            off = cid * ec + sid * ess + g * G
            pltpu.async_copy(r_hbm.at[pl.ds(off, G)], ibuf, si)
            pltpu.async_copy(e_hbm.at[pl.ds(off, G)], rbuf, sr)

        def finish(g, ibuf, rbuf, si, sr):
            off = cid * ec + sid * ess + g * G
            pltpu.make_async_copy(r_hbm.at[pl.ds(off, G)], ibuf, si).wait()
            pltpu.make_async_copy(e_hbm.at[pl.ds(off, G)], rbuf, sr).wait()
            pltpu.sync_copy(rbuf, acc_sh.at[ibuf], add=True)

        start(0, idx_a, rows_a, sem_ia, sem_ra)

        @pl.loop(0, npairs)
        def _(i):
            start(2 * i + 1, idx_b, rows_b, sem_ib, sem_rb)
            finish(2 * i, idx_a, rows_a, sem_ia, sem_ra)

            @pl.when(2 * i + 2 < ng)
            def _():
                start(2 * i + 2, idx_a, rows_a, sem_ia, sem_ra)

            finish(2 * i + 1, idx_b, rows_b, sem_ib, sem_rb)

        if ng % 2 == 1:
            finish(ng - 1, idx_a, rows_a, sem_ia, sem_ra)

        plsc.subcore_barrier()

        @pl.loop(0, zrounds)
        def _(j):
            gi = j * NS + sid

            @pl.when(gi < zg)
            def _():
                pltpu.sync_copy(acc_sh.at[pl.ds(gi * G, G)],
                                out_hbm.at[pl.ds(cid * N + gi * G, G)])

    return k(e_new, receivers, init)


def _ln(y, g, bl):
    mu = jnp.mean(y, axis=-1, keepdims=True)
    var = jnp.mean((y - mu) * (y - mu), axis=-1, keepdims=True)
    return (y - mu) * lax.rsqrt(var + 1e-5) * g + bl


def _dot(a, b):
    return jnp.dot(a, b, preferred_element_type=jnp.float32)


def _full_spec(shape):
    return pl.BlockSpec(shape, lambda i: tuple(0 for _ in shape))


def _mlp_ln_kernel(x_ref, w1_ref, b1_ref, w2_ref, b2_ref, g_ref, bl_ref,
                   o_ref):
    z = jnp.maximum(_dot(x_ref[...], w1_ref[...]) + b1_ref[...], 0.0)
    y = _dot(z, w2_ref[...]) + b2_ref[...]
    o_ref[...] = _ln(y, g_ref[...], bl_ref[...])


def _encode(x, p, tile):
    n, d = x.shape
    assert n % tile == 0
    H = p["w1"].shape[1]
    return pl.pallas_call(
        _mlp_ln_kernel,
        grid=(n // tile,),
        in_specs=[
            pl.BlockSpec((tile, d), lambda i: (i, 0)),
            _full_spec(p["w1"].shape),
            _full_spec((1, H)),
            _full_spec(p["w2"].shape),
            _full_spec((1, p["w2"].shape[1])),
            _full_spec((1, p["w2"].shape[1])),
            _full_spec((1, p["w2"].shape[1])),
        ],
        out_specs=pl.BlockSpec((tile, p["w2"].shape[1]), lambda i: (i, 0)),
        out_shape=jax.ShapeDtypeStruct((n, p["w2"].shape[1]), jnp.float32),
    )(x, p["w1"], p["b1"].reshape(1, -1), p["w2"], p["b2"].reshape(1, -1),
      p["g"].reshape(1, -1), p["bl"].reshape(1, -1))


def _proj_kernel(h_ref, ws_ref, wr_ref, ps_ref, pr_ref):
    ps_ref[...] = _dot(h_ref[...], ws_ref[...])
    pr_ref[...] = _dot(h_ref[...], wr_ref[...])


def _project(h, ws, wr, tile):
    N, H = h.shape
    assert N % tile == 0
    row = pl.BlockSpec((tile, H), lambda i: (i, 0))
    out = jax.ShapeDtypeStruct((N, H), jnp.float32)
    return pl.pallas_call(
        _proj_kernel,
        grid=(N // tile,),
        in_specs=[row, _full_spec((H, H)), _full_spec((H, H))],
        out_specs=(row, row),
        out_shape=(out, out),
    )(h, ws, wr)


def _edge_kernel(gs_ref, e_ref, w1e_ref, b1_ref, w2_ref, b2_ref,
                 g_ref, bl_ref, enew_ref, enext_ref):
    z = (gs_ref[...] + _dot(e_ref[...], w1e_ref[...]) + b1_ref[...])
    z = jnp.maximum(z, 0.0)
    y = _dot(z, w2_ref[...]) + b2_ref[...]
    e_new = _ln(y, g_ref[...], bl_ref[...])
    enew_ref[...] = e_new
    enext_ref[...] = e_ref[...] + e_new


def _edge_mlp(gsum, e, p, tile):
    E, H = e.shape
    assert E % tile == 0
    out = jax.ShapeDtypeStruct((E, H), jnp.float32)
    row = pl.BlockSpec((tile, H), lambda i: (i, 0))
    return pl.pallas_call(
        _edge_kernel,
        grid=(E // tile,),
        in_specs=[row, row,
                  _full_spec((H, H)), _full_spec((1, H)),
                  _full_spec((H, H)), _full_spec((1, H)),
                  _full_spec((1, H)), _full_spec((1, H))],
        out_specs=(row, row),
        out_shape=(out, out),
    )(gsum, e, p["w1"][2 * H:3 * H], p["b1"].reshape(1, -1), p["w2"],
      p["b2"].reshape(1, -1), p["g"].reshape(1, -1), p["bl"].reshape(1, -1))


def _node_kernel(h_ref, a0_ref, a1_ref, w1_ref, b1_ref, w2_ref, b2_ref,
                 g_ref, bl_ref, o_ref):
    H = h_ref.shape[1]
    agg = a0_ref[...] + a1_ref[...]
    z = (_dot(h_ref[...], w1_ref[0:H])
         + _dot(agg, w1_ref[H:2 * H])
         + b1_ref[...])
    z = jnp.maximum(z, 0.0)
    y = _dot(z, w2_ref[...]) + b2_ref[...]
    o_ref[...] = h_ref[...] + _ln(y, g_ref[...], bl_ref[...])


def _node_mlp(h, partials, p, tile):
    N, H = h.shape
    assert N % tile == 0
    nb = N // tile
    row = pl.BlockSpec((tile, H), lambda i: (i, 0))
    return pl.pallas_call(
        _node_kernel,
        grid=(nb,),
        in_specs=[row,
                  pl.BlockSpec((tile, H), lambda i: (i, 0)),
                  pl.BlockSpec((tile, H), lambda i: (i + nb, 0)),
                  _full_spec((2 * H, H)), _full_spec((1, H)),
                  _full_spec((H, H)), _full_spec((1, H)),
                  _full_spec((1, H)), _full_spec((1, H))],
        out_specs=row,
        out_shape=jax.ShapeDtypeStruct((N, H), jnp.float32),
    )(h, partials, partials, p["w1"], p["b1"].reshape(1, -1), p["w2"],
      p["b2"].reshape(1, -1), p["g"].reshape(1, -1), p["bl"].reshape(1, -1))


def _dec_kernel(h_ref, w1_ref, b1_ref, w2_ref, b2_ref, o_ref):
    z = jnp.maximum(_dot(h_ref[...], w1_ref[...]) + b1_ref[...], 0.0)
    o_ref[...] = _dot(z, w2_ref[...]) + b2_ref[...]


def _decode(h, p, tile):
    N, H = h.shape
    out_d = p["w2"].shape[1]
    return pl.pallas_call(
        _dec_kernel,
        grid=(N // tile,),
        in_specs=[pl.BlockSpec((tile, H), lambda i: (i, 0)),
                  _full_spec((H, H)), _full_spec((1, H)),
                  _full_spec((H, out_d)), _full_spec((1, out_d))],
        out_specs=pl.BlockSpec((tile, out_d), lambda i: (i, 0)),
        out_shape=jax.ShapeDtypeStruct((N, out_d), jnp.float32),
    )(h, p["w1"], p["b1"].reshape(1, -1), p["w2"], p["b2"].reshape(1, -1))


def kernel(x, edge_index, edge_attr, params):
    N = x.shape[0]
    E = edge_attr.shape[0]
    H = params["enc_node"]["w2"].shape[1]
    senders = edge_index[0]
    receivers = edge_index[1]

    h = _encode(x, params["enc_node"], tile=2000)
    e = _encode(edge_attr, params["enc_edge"], tile=2560)
    zeros = jnp.zeros((NC * N, H), jnp.float32)

    NCHUNK = 2
    unit = NW * G
    groups = E // unit
    per = groups // NCHUNK
    sizes = [per * unit] * (NCHUNK - 1)
    sizes.append(E - sum(sizes))
    offs = [sum(sizes[:i]) for i in range(NCHUNK)]
    sch = [senders[o:o + n] for o, n in zip(offs, sizes)]
    rch = [receivers[o:o + n] for o, n in zip(offs, sizes)]
    ech = [e[o:o + n] for o, n in zip(offs, sizes)]

    for blk in params["blocks"]:
        w1 = blk["edge"]["w1"]
        ps, pr = _project(h, w1[0:H], w1[H:2 * H], tile=2000)
        gs = [_sc_gather_sum(ps, pr, s, r) for s, r in zip(sch, rch)]
        en = []
        for i in range(NCHUNK):
            e_new, ech[i] = _edge_mlp(gs[i], ech[i], blk["edge"], tile=2560)
            en.append(e_new)
        p = zeros
        for i in range(NCHUNK):
            p = _sc_scatter_add(en[i], rch[i], p)
        h = _node_mlp(h, p, blk["node"], tile=2000)

    return _decode(h, params["dec"], tile=2000)


# speedup vs baseline: 1.2602x; 1.0035x over previous
import functools

import jax
import jax.numpy as jnp
from jax import lax
from jax.experimental import pallas as pl
from jax.experimental.pallas import tpu as pltpu
from jax.experimental.pallas import tpu_sc as plsc

NC = 2
NS = 16
NW = NC * NS
G = 80

def _sc_mesh():
    return plsc.VectorSubcoreMesh(core_axis_name="c", subcore_axis_name="s")


def _sc_gather_sum(ps, pr, senders, receivers):
    E = senders.shape[0]
    Hd = ps.shape[1]
    assert E % (NW * G) == 0
    ew = E // NW
    ng = ew // G
    assert ng >= 2 and G % 16 == 0
    npairs = ng // 2

    @functools.partial(
        pl.kernel,
        mesh=_sc_mesh(),
        out_type=jax.ShapeDtypeStruct((E, Hd), jnp.float32),
        scratch_types=[
            pltpu.VMEM_SHARED((2 * NS * G, Hd), jnp.float32),
            pltpu.VMEM((ew,), jnp.int32),
            pltpu.VMEM((ew,), jnp.int32),
            pltpu.VMEM((G,), jnp.int32),
            pltpu.VMEM((G,), jnp.int32),
            pltpu.VMEM((G, Hd), jnp.float32),
            pltpu.VMEM((G, Hd), jnp.float32),
            pltpu.VMEM((G, Hd), jnp.float32),
            pltpu.VMEM((G, Hd), jnp.float32),
            pltpu.SemaphoreType.DMA,
            pltpu.SemaphoreType.DMA,
            pltpu.SemaphoreType.DMA,
            pltpu.SemaphoreType.DMA,
        ],
    )
    def k(ps_hbm, pr_hbm, s_hbm, r_hbm, gs_hbm, sh, si, ri, idv_a, idv_b,
          s_a, s_b, r_a, r_b, sem_sa, sem_ra, sem_sb, sem_rb):
        cid = lax.axis_index("c")
        sid = lax.axis_index("s")
        wid = sid * NC + cid
        base = wid * ew
        slot_a = sid * 2 * G
        slot_b = slot_a + G
        pltpu.sync_copy(s_hbm.at[pl.ds(base, ew)], si)
        pltpu.sync_copy(r_hbm.at[pl.ds(base, ew)], ri)
        for c in range(G // 16):
            chunk = lax.iota(jnp.int32, 16) + (16 * c + slot_a)
            idv_a[pl.ds(16 * c, 16)] = chunk
            idv_b[pl.ds(16 * c, 16)] = chunk + G

        def start(g, sbuf, rbuf, ss, sr):
            pltpu.async_copy(ps_hbm.at[si.at[pl.ds(g * G, G)]], sbuf, ss)
            pltpu.async_copy(pr_hbm.at[ri.at[pl.ds(g * G, G)]], rbuf, sr)

        def finish(g, slot, idv, sbuf, rbuf, ss, sr):
            pltpu.make_async_copy(ps_hbm.at[si.at[pl.ds(g * G, G)]], sbuf,
                                  ss).wait()
            pltpu.make_async_copy(pr_hbm.at[ri.at[pl.ds(g * G, G)]], rbuf,
                                  sr).wait()
            pltpu.sync_copy(sbuf, sh.at[pl.ds(slot, G)])
            pltpu.sync_copy(rbuf, sh.at[idv], add=True)
            pltpu.sync_copy(sh.at[pl.ds(slot, G)],
                            gs_hbm.at[pl.ds(base + g * G, G)])

        start(0, s_a, r_a, sem_sa, sem_ra)

        @pl.loop(0, npairs)
        def _(i):
            start(2 * i + 1, s_b, r_b, sem_sb, sem_rb)
            finish(2 * i, slot_a, idv_a, s_a, r_a, sem_sa, sem_ra)

            @pl.when(2 * i + 2 < ng)
            def _():
                start(2 * i + 2, s_a, r_a, sem_sa, sem_ra)

            finish(2 * i + 1, slot_b, idv_b, s_b, r_b, sem_sb, sem_rb)

        if ng % 2 == 1:
            finish(ng - 1, slot_a, idv_a, s_a, r_a, sem_sa, sem_ra)

    return k(ps, pr, senders, receivers)


def _sc_scatter_add(e_new, receivers, init):
    E, Hd = e_new.shape
    N = init.shape[0] // NC
    assert E % (NC * NS * G) == 0
    ec = E // NC
    ess = ec // NS
    ng = ess // G
    zg = -(-N // G)
    assert N % G == 0
    zrounds = -(-zg // NS)

    assert ng >= 2
    npairs = ng // 2

    @functools.partial(
        pl.kernel,
        mesh=_sc_mesh(),
        out_type=jax.ShapeDtypeStruct((NC * N, Hd), jnp.float32),
        scratch_types=[
            pltpu.VMEM_SHARED((N, Hd), jnp.float32),
            pltpu.VMEM((G,), jnp.int32),
            pltpu.VMEM((G,), jnp.int32),
            pltpu.VMEM((G, Hd), jnp.float32),
            pltpu.VMEM((G, Hd), jnp.float32),
            pltpu.SemaphoreType.DMA,
            pltpu.SemaphoreType.DMA,
            pltpu.SemaphoreType.DMA,
            pltpu.SemaphoreType.DMA,
        ],
    )
    def k(e_hbm, r_hbm, z_hbm, out_hbm, acc_sh, idx_a, idx_b, rows_a, rows_b,
          sem_ia, sem_ra, sem_ib, sem_rb):
        cid = lax.axis_index("c")
        sid = lax.axis_index("s")

        @pl.loop(0, zrounds)
        def _(j):
            gi = j * NS + sid

            @pl.when(gi < zg)
            def _():
                pltpu.async_copy(z_hbm.at[pl.ds(cid * N + gi * G, G)],
                                 acc_sh.at[pl.ds(gi * G, G)], sem_ib)

        def start(g, ibuf, rbuf, si, sr):
            off = cid * ec + sid * ess + g * G
            pltpu.async_copy(r_hbm.at[pl.ds(off, G)], ibuf, si)
            pltpu.async_copy(e_hbm.at[pl.ds(off, G)], rbuf, sr)

        def finish(g, ibuf, rbuf, si, sr):
            off = cid * ec + sid * ess + g * G
            pltpu.make_async_copy(r_hbm.at[pl.ds(off, G)], ibuf, si).wait()
            pltpu.make_async_copy(e_hbm.at[pl.ds(off, G)], rbuf, sr).wait()
            pltpu.sync_copy(rbuf, acc_sh.at[ibuf], add=True)

        start(0, idx_a, rows_a, sem_ia, sem_ra)

        @pl.loop(0, zrounds)
        def _(j):
            gi = j * NS + sid

            @pl.when(gi < zg)
            def _():
                pltpu.make_async_copy(z_hbm.at[pl.ds(cid * N + gi * G, G)],
                                      acc_sh.at[pl.ds(gi * G, G)],
                                      sem_ib).wait()

        plsc.subcore_barrier()

        @pl.loop(0, npairs)
        def _(i):
            start(2 * i + 1, idx_b, rows_b, sem_ib, sem_rb)
            finish(2 * i, idx_a, rows_a, sem_ia, sem_ra)

            @pl.when(2 * i + 2 < ng)
            def _():
                start(2 * i + 2, idx_a, rows_a, sem_ia, sem_ra)

            finish(2 * i + 1, idx_b, rows_b, sem_ib, sem_rb)

        if ng % 2 == 1:
            finish(ng - 1, idx_a, rows_a, sem_ia, sem_ra)

        plsc.subcore_barrier()

        @pl.loop(0, zrounds)
        def _(j):
            gi = j * NS + sid

            @pl.when(gi < zg)
            def _():
                pltpu.async_copy(acc_sh.at[pl.ds(gi * G, G)],
                                 out_hbm.at[pl.ds(cid * N + gi * G, G)],
                                 sem_ia)

        @pl.loop(0, zrounds)
        def _(j):
            gi = j * NS + sid

            @pl.when(gi < zg)
            def _():
                pltpu.make_async_copy(acc_sh.at[pl.ds(gi * G, G)],
                                      out_hbm.at[pl.ds(cid * N + gi * G, G)],
                                      sem_ia).wait()

    return k(e_new, receivers, init)


def _ln(y, g, bl):
    mu = jnp.mean(y, axis=-1, keepdims=True)
    var = jnp.mean((y - mu) * (y - mu), axis=-1, keepdims=True)
    return (y - mu) * lax.rsqrt(var + 1e-5) * g + bl


def _dot(a, b):
    return jnp.dot(a, b, preferred_element_type=jnp.float32)


def _full_spec(shape):
    return pl.BlockSpec(shape, lambda i: tuple(0 for _ in shape))


def _mlp_ln_kernel(x_ref, w1_ref, b1_ref, w2_ref, b2_ref, g_ref, bl_ref,
                   o_ref):
    z = jnp.maximum(_dot(x_ref[...], w1_ref[...]) + b1_ref[...], 0.0)
    y = _dot(z, w2_ref[...]) + b2_ref[...]
    o_ref[...] = _ln(y, g_ref[...], bl_ref[...])


def _encode(x, p, tile):
    n, d = x.shape
    assert n % tile == 0
    H = p["w1"].shape[1]
    return pl.pallas_call(
        _mlp_ln_kernel,
        grid=(n // tile,),
        in_specs=[
            pl.BlockSpec((tile, d), lambda i: (i, 0)),
            _full_spec(p["w1"].shape),
            _full_spec((1, H)),
            _full_spec(p["w2"].shape),
            _full_spec((1, p["w2"].shape[1])),
            _full_spec((1, p["w2"].shape[1])),
            _full_spec((1, p["w2"].shape[1])),
        ],
        out_specs=pl.BlockSpec((tile, p["w2"].shape[1]), lambda i: (i, 0)),
        out_shape=jax.ShapeDtypeStruct((n, p["w2"].shape[1]), jnp.float32),
    )(x, p["w1"], p["b1"].reshape(1, -1), p["w2"], p["b2"].reshape(1, -1),
      p["g"].reshape(1, -1), p["bl"].reshape(1, -1))


def _proj_kernel(h_ref, ws_ref, wr_ref, ps_ref, pr_ref):
    ps_ref[...] = _dot(h_ref[...], ws_ref[...])
    pr_ref[...] = _dot(h_ref[...], wr_ref[...])


def _project(h, ws, wr, tile):
    N, H = h.shape
    assert N % tile == 0
    row = pl.BlockSpec((tile, H), lambda i: (i, 0))
    out = jax.ShapeDtypeStruct((N, H), jnp.float32)
    return pl.pallas_call(
        _proj_kernel,
        grid=(N // tile,),
        in_specs=[row, _full_spec((H, H)), _full_spec((H, H))],
        out_specs=(row, row),
        out_shape=(out, out),
    )(h, ws, wr)


def _edge_kernel(gs_ref, e_ref, w1e_ref, b1_ref, w2_ref, b2_ref,
                 g_ref, bl_ref, enew_ref, enext_ref):
    z = (gs_ref[...] + _dot(e_ref[...], w1e_ref[...]) + b1_ref[...])
    z = jnp.maximum(z, 0.0)
    y = _dot(z, w2_ref[...]) + b2_ref[...]
    e_new = _ln(y, g_ref[...], bl_ref[...])
    enew_ref[...] = e_new
    enext_ref[...] = e_ref[...] + e_new


def _edge_mlp(gsum, e, p, tile):
    E, H = e.shape
    assert E % tile == 0
    out = jax.ShapeDtypeStruct((E, H), jnp.float32)
    row = pl.BlockSpec((tile, H), lambda i: (i, 0))
    return pl.pallas_call(
        _edge_kernel,
        grid=(E // tile,),
        in_specs=[row, row,
                  _full_spec((H, H)), _full_spec((1, H)),
                  _full_spec((H, H)), _full_spec((1, H)),
                  _full_spec((1, H)), _full_spec((1, H))],
        out_specs=(row, row),
        out_shape=(out, out),
    )(gsum, e, p["w1"][2 * H:3 * H], p["b1"].reshape(1, -1), p["w2"],
      p["b2"].reshape(1, -1), p["g"].reshape(1, -1), p["bl"].reshape(1, -1))


def _node_kernel(h_ref, a0_ref, a1_ref, w1_ref, b1_ref, w2_ref, b2_ref,
                 g_ref, bl_ref, o_ref):
    H = h_ref.shape[1]
    agg = a0_ref[...] + a1_ref[...]
    z = (_dot(h_ref[...], w1_ref[0:H])
         + _dot(agg, w1_ref[H:2 * H])
         + b1_ref[...])
    z = jnp.maximum(z, 0.0)
    y = _dot(z, w2_ref[...]) + b2_ref[...]
    o_ref[...] = h_ref[...] + _ln(y, g_ref[...], bl_ref[...])


def _node_mlp(h, partials, p, tile):
    N, H = h.shape
    assert N % tile == 0
    nb = N // tile
    row = pl.BlockSpec((tile, H), lambda i: (i, 0))
    return pl.pallas_call(
        _node_kernel,
        grid=(nb,),
        in_specs=[row,
                  pl.BlockSpec((tile, H), lambda i: (i, 0)),
                  pl.BlockSpec((tile, H), lambda i: (i + nb, 0)),
                  _full_spec((2 * H, H)), _full_spec((1, H)),
                  _full_spec((H, H)), _full_spec((1, H)),
                  _full_spec((1, H)), _full_spec((1, H))],
        out_specs=row,
        out_shape=jax.ShapeDtypeStruct((N, H), jnp.float32),
    )(h, partials, partials, p["w1"], p["b1"].reshape(1, -1), p["w2"],
      p["b2"].reshape(1, -1), p["g"].reshape(1, -1), p["bl"].reshape(1, -1))


def _dec_kernel(h_ref, w1_ref, b1_ref, w2_ref, b2_ref, o_ref):
    z = jnp.maximum(_dot(h_ref[...], w1_ref[...]) + b1_ref[...], 0.0)
    o_ref[...] = _dot(z, w2_ref[...]) + b2_ref[...]


def _decode(h, p, tile):
    N, H = h.shape
    out_d = p["w2"].shape[1]
    return pl.pallas_call(
        _dec_kernel,
        grid=(N // tile,),
        in_specs=[pl.BlockSpec((tile, H), lambda i: (i, 0)),
                  _full_spec((H, H)), _full_spec((1, H)),
                  _full_spec((H, out_d)), _full_spec((1, out_d))],
        out_specs=pl.BlockSpec((tile, out_d), lambda i: (i, 0)),
        out_shape=jax.ShapeDtypeStruct((N, out_d), jnp.float32),
    )(h, p["w1"], p["b1"].reshape(1, -1), p["w2"], p["b2"].reshape(1, -1))


def kernel(x, edge_index, edge_attr, params):
    N = x.shape[0]
    E = edge_attr.shape[0]
    H = params["enc_node"]["w2"].shape[1]
    senders = edge_index[0]
    receivers = edge_index[1]

    h = _encode(x, params["enc_node"], tile=2000)
    e = _encode(edge_attr, params["enc_edge"], tile=2560)
    zeros = jnp.zeros((NC * N, H), jnp.float32)

    NCHUNK = 2
    unit = NW * G
    groups = E // unit
    per = groups // NCHUNK
    sizes = [per * unit] * (NCHUNK - 1)
    sizes.append(E - sum(sizes))
    offs = [sum(sizes[:i]) for i in range(NCHUNK)]
    sch = [senders[o:o + n] for o, n in zip(offs, sizes)]
    rch = [receivers[o:o + n] for o, n in zip(offs, sizes)]
    ech = [e[o:o + n] for o, n in zip(offs, sizes)]

    for blk in params["blocks"]:
        w1 = blk["edge"]["w1"]
        ps, pr = _project(h, w1[0:H], w1[H:2 * H], tile=2000)
        gs = [_sc_gather_sum(ps, pr, s, r) for s, r in zip(sch, rch)]
        en = []
        for i in range(NCHUNK):
            e_new, ech[i] = _edge_mlp(gs[i], ech[i], blk["edge"], tile=2560)
            en.append(e_new)
        p = zeros
        for i in range(NCHUNK):
            p = _sc_scatter_add(en[i], rch[i], p)
        h = _node_mlp(h, p, blk["node"], tile=2000)

    return _decode(h, params["dec"], tile=2000)


# speedup vs baseline: 1.2862x; 1.0207x over previous
import functools

import jax
import jax.numpy as jnp
from jax import lax
from jax.experimental import pallas as pl
from jax.experimental.pallas import tpu as pltpu
from jax.experimental.pallas import tpu_sc as plsc

NC = 2
NS = 16
NW = NC * NS
G = 80

def _sc_mesh():
    return plsc.VectorSubcoreMesh(core_axis_name="c", subcore_axis_name="s")


def _sc_gather_sum(ps, pr, senders, receivers):
    E = senders.shape[0]
    Hd = ps.shape[1]
    assert E % (NW * G) == 0
    ew = E // NW
    ng = ew // G
    assert ng >= 2 and G % 16 == 0
    npairs = ng // 2

    @functools.partial(
        pl.kernel,
        mesh=_sc_mesh(),
        out_type=jax.ShapeDtypeStruct((E, Hd), jnp.float32),
        scratch_types=[
            pltpu.VMEM_SHARED((2 * NS * G, Hd), jnp.float32),
            pltpu.VMEM((ew,), jnp.int32),
            pltpu.VMEM((ew,), jnp.int32),
            pltpu.VMEM((G,), jnp.int32),
            pltpu.VMEM((G,), jnp.int32),
            pltpu.VMEM((G, Hd), jnp.float32),
            pltpu.VMEM((G, Hd), jnp.float32),
            pltpu.VMEM((G, Hd), jnp.float32),
            pltpu.VMEM((G, Hd), jnp.float32),
            pltpu.SemaphoreType.DMA,
            pltpu.SemaphoreType.DMA,
            pltpu.SemaphoreType.DMA,
            pltpu.SemaphoreType.DMA,
            pltpu.SemaphoreType.DMA,
            pltpu.SemaphoreType.DMA,
        ],
    )
    def k(ps_hbm, pr_hbm, s_hbm, r_hbm, gs_hbm, sh, si, ri, idv_a, idv_b,
          s_a, s_b, r_a, r_b, sem_sa, sem_ra, sem_sb, sem_rb, sem_wa,
          sem_wb):
        cid = lax.axis_index("c")
        sid = lax.axis_index("s")
        wid = sid * NC + cid
        base = wid * ew
        slot_a = sid * 2 * G
        slot_b = slot_a + G
        pltpu.sync_copy(s_hbm.at[pl.ds(base, ew)], si)
        pltpu.sync_copy(r_hbm.at[pl.ds(base, ew)], ri)
        for c in range(G // 16):
            chunk = lax.iota(jnp.int32, 16) + (16 * c + slot_a)
            idv_a[pl.ds(16 * c, 16)] = chunk
            idv_b[pl.ds(16 * c, 16)] = chunk + G

        def start(g, sbuf, rbuf, ss, sr):
            pltpu.async_copy(ps_hbm.at[si.at[pl.ds(g * G, G)]], sbuf, ss)
            pltpu.async_copy(pr_hbm.at[ri.at[pl.ds(g * G, G)]], rbuf, sr)

        def finish(g, slot, idv, sbuf, rbuf, ss, sr, sw):
            pltpu.make_async_copy(ps_hbm.at[si.at[pl.ds(g * G, G)]], sbuf,
                                  ss).wait()
            pltpu.make_async_copy(pr_hbm.at[ri.at[pl.ds(g * G, G)]], rbuf,
                                  sr).wait()

            @pl.when(g >= 2)
            def _():
                pltpu.make_async_copy(
                    sh.at[pl.ds(slot, G)],
                    gs_hbm.at[pl.ds(base + (g - 2) * G, G)], sw).wait()

            pltpu.sync_copy(sbuf, sh.at[pl.ds(slot, G)])
            pltpu.sync_copy(rbuf, sh.at[idv], add=True)
            pltpu.async_copy(sh.at[pl.ds(slot, G)],
                             gs_hbm.at[pl.ds(base + g * G, G)], sw)

        start(0, s_a, r_a, sem_sa, sem_ra)

        @pl.loop(0, npairs)
        def _(i):
            start(2 * i + 1, s_b, r_b, sem_sb, sem_rb)
            finish(2 * i, slot_a, idv_a, s_a, r_a, sem_sa, sem_ra, sem_wa)

            @pl.when(2 * i + 2 < ng)
            def _():
                start(2 * i + 2, s_a, r_a, sem_sa, sem_ra)

            finish(2 * i + 1, slot_b, idv_b, s_b, r_b, sem_sb, sem_rb,
                   sem_wb)

        if ng % 2 == 1:
            finish(ng - 1, slot_a, idv_a, s_a, r_a, sem_sa, sem_ra, sem_wa)
            pltpu.make_async_copy(sh.at[pl.ds(slot_a, G)],
                                  gs_hbm.at[pl.ds(base + (ng - 1) * G, G)],
                                  sem_wa).wait()
            pltpu.make_async_copy(sh.at[pl.ds(slot_b, G)],
                                  gs_hbm.at[pl.ds(base + (ng - 2) * G, G)],
                                  sem_wb).wait()
        else:
            pltpu.make_async_copy(sh.at[pl.ds(slot_a, G)],
                                  gs_hbm.at[pl.ds(base + (ng - 2) * G, G)],
                                  sem_wa).wait()
            pltpu.make_async_copy(sh.at[pl.ds(slot_b, G)],
                                  gs_hbm.at[pl.ds(base + (ng - 1) * G, G)],
                                  sem_wb).wait()

    return k(ps, pr, senders, receivers)


def _sc_scatter_add(e_new, receivers, init):
    E, Hd = e_new.shape
    N = init.shape[0] // NC
    assert E % (NC * NS * G) == 0
    ec = E // NC
    ess = ec // NS
    ng = ess // G
    zg = -(-N // G)
    assert N % G == 0
    zrounds = -(-zg // NS)

    assert ng >= 2
    npairs = ng // 2

    @functools.partial(
        pl.kernel,
        mesh=_sc_mesh(),
        out_type=jax.ShapeDtypeStruct((NC * N, Hd), jnp.float32),
        scratch_types=[
            pltpu.VMEM_SHARED((N, Hd), jnp.float32),
            pltpu.VMEM((G,), jnp.int32),
            pltpu.VMEM((G,), jnp.int32),
            pltpu.VMEM((G, Hd), jnp.float32),
            pltpu.VMEM((G, Hd), jnp.float32),
            pltpu.SemaphoreType.DMA,
            pltpu.SemaphoreType.DMA,
            pltpu.SemaphoreType.DMA,
            pltpu.SemaphoreType.DMA,
        ],
    )
    def k(e_hbm, r_hbm, z_hbm, out_hbm, acc_sh, idx_a, idx_b, rows_a, rows_b,
          sem_ia, sem_ra, sem_ib, sem_rb):
        cid = lax.axis_index("c")
        sid = lax.axis_index("s")

        @pl.loop(0, zrounds)
        def _(j):
            gi = j * NS + sid

            @pl.when(gi < zg)
            def _():
                pltpu.async_copy(z_hbm.at[pl.ds(cid * N + gi * G, G)],
                                 acc_sh.at[pl.ds(gi * G, G)], sem_ib)

        def start(g, ibuf, rbuf, si, sr):
            off = cid * ec + sid * ess + g * G
            pltpu.async_copy(r_hbm.at[pl.ds(off, G)], ibuf, si)
            pltpu.async_copy(e_hbm.at[pl.ds(off, G)], rbuf, sr)

        def finish(g, ibuf, rbuf, si, sr):
            off = cid * ec + sid * ess + g * G
            pltpu.make_async_copy(r_hbm.at[pl.ds(off, G)], ibuf, si).wait()
            pltpu.make_async_copy(e_hbm.at[pl.ds(off, G)], rbuf, sr).wait()
            pltpu.sync_copy(rbuf, acc_sh.at[ibuf], add=True)

        start(0, idx_a, rows_a, sem_ia, sem_ra)

        @pl.loop(0, zrounds)
        def _(j):
            gi = j * NS + sid

            @pl.when(gi < zg)
            def _():
                pltpu.make_async_copy(z_hbm.at[pl.ds(cid * N + gi * G, G)],
                                      acc_sh.at[pl.ds(gi * G, G)],
                                      sem_ib).wait()

        plsc.subcore_barrier()

        @pl.loop(0, npairs)
        def _(i):
            start(2 * i + 1, idx_b, rows_b, sem_ib, sem_rb)
            finish(2 * i, idx_a, rows_a, sem_ia, sem_ra)

            @pl.when(2 * i + 2 < ng)
            def _():
                start(2 * i + 2, idx_a, rows_a, sem_ia, sem_ra)

            finish(2 * i + 1, idx_b, rows_b, sem_ib, sem_rb)

        if ng % 2 == 1:
            finish(ng - 1, idx_a, rows_a, sem_ia, sem_ra)

        plsc.subcore_barrier()

        @pl.loop(0, zrounds)
        def _(j):
            gi = j * NS + sid

            @pl.when(gi < zg)
            def _():
                pltpu.async_copy(acc_sh.at[pl.ds(gi * G, G)],
                                 out_hbm.at[pl.ds(cid * N + gi * G, G)],
                                 sem_ia)

        @pl.loop(0, zrounds)
        def _(j):
            gi = j * NS + sid

            @pl.when(gi < zg)
            def _():
                pltpu.make_async_copy(acc_sh.at[pl.ds(gi * G, G)],
                                      out_hbm.at[pl.ds(cid * N + gi * G, G)],
                                      sem_ia).wait()

    return k(e_new, receivers, init)


def _ln(y, g, bl):
    mu = jnp.mean(y, axis=-1, keepdims=True)
    var = jnp.mean((y - mu) * (y - mu), axis=-1, keepdims=True)
    return (y - mu) * lax.rsqrt(var + 1e-5) * g + bl


def _dot(a, b):
    return jnp.dot(a, b, preferred_element_type=jnp.float32)


def _full_spec(shape):
    return pl.BlockSpec(shape, lambda i: tuple(0 for _ in shape))


def _mlp_ln_kernel(x_ref, w1_ref, b1_ref, w2_ref, b2_ref, g_ref, bl_ref,
                   o_ref):
    z = jnp.maximum(_dot(x_ref[...], w1_ref[...]) + b1_ref[...], 0.0)
    y = _dot(z, w2_ref[...]) + b2_ref[...]
    o_ref[...] = _ln(y, g_ref[...], bl_ref[...])


def _encode(x, p, tile):
    n, d = x.shape
    assert n % tile == 0
    H = p["w1"].shape[1]
    return pl.pallas_call(
        _mlp_ln_kernel,
        grid=(n // tile,),
        in_specs=[
            pl.BlockSpec((tile, d), lambda i: (i, 0)),
            _full_spec(p["w1"].shape),
            _full_spec((1, H)),
            _full_spec(p["w2"].shape),
            _full_spec((1, p["w2"].shape[1])),
            _full_spec((1, p["w2"].shape[1])),
            _full_spec((1, p["w2"].shape[1])),
        ],
        out_specs=pl.BlockSpec((tile, p["w2"].shape[1]), lambda i: (i, 0)),
        out_shape=jax.ShapeDtypeStruct((n, p["w2"].shape[1]), jnp.float32),
    )(x, p["w1"], p["b1"].reshape(1, -1), p["w2"], p["b2"].reshape(1, -1),
      p["g"].reshape(1, -1), p["bl"].reshape(1, -1))


def _proj_kernel(h_ref, ws_ref, wr_ref, ps_ref, pr_ref):
    ps_ref[...] = _dot(h_ref[...], ws_ref[...])
    pr_ref[...] = _dot(h_ref[...], wr_ref[...])


def _project(h, ws, wr, tile):
    N, H = h.shape
    assert N % tile == 0
    row = pl.BlockSpec((tile, H), lambda i: (i, 0))
    out = jax.ShapeDtypeStruct((N, H), jnp.float32)
    return pl.pallas_call(
        _proj_kernel,
        grid=(N // tile,),
        in_specs=[row, _full_spec((H, H)), _full_spec((H, H))],
        out_specs=(row, row),
        out_shape=(out, out),
    )(h, ws, wr)


def _edge_kernel(gs_ref, e_ref, w1e_ref, b1_ref, w2_ref, b2_ref,
                 g_ref, bl_ref, enew_ref, enext_ref):
    z = (gs_ref[...] + _dot(e_ref[...], w1e_ref[...]) + b1_ref[...])
    z = jnp.maximum(z, 0.0)
    y = _dot(z, w2_ref[...]) + b2_ref[...]
    e_new = _ln(y, g_ref[...], bl_ref[...])
    enew_ref[...] = e_new
    enext_ref[...] = e_ref[...] + e_new


def _edge_mlp(gsum, e, p, tile):
    E, H = e.shape
    assert E % tile == 0
    out = jax.ShapeDtypeStruct((E, H), jnp.float32)
    row = pl.BlockSpec((tile, H), lambda i: (i, 0))
    return pl.pallas_call(
        _edge_kernel,
        grid=(E // tile,),
        in_specs=[row, row,
                  _full_spec((H, H)), _full_spec((1, H)),
                  _full_spec((H, H)), _full_spec((1, H)),
                  _full_spec((1, H)), _full_spec((1, H))],
        out_specs=(row, row),
        out_shape=(out, out),
    )(gsum, e, p["w1"][2 * H:3 * H], p["b1"].reshape(1, -1), p["w2"],
      p["b2"].reshape(1, -1), p["g"].reshape(1, -1), p["bl"].reshape(1, -1))


def _node_kernel(h_ref, a0_ref, a1_ref, w1_ref, b1_ref, w2_ref, b2_ref,
                 g_ref, bl_ref, o_ref):
    H = h_ref.shape[1]
    agg = a0_ref[...] + a1_ref[...]
    z = (_dot(h_ref[...], w1_ref[0:H])
         + _dot(agg, w1_ref[H:2 * H])
         + b1_ref[...])
    z = jnp.maximum(z, 0.0)
    y = _dot(z, w2_ref[...]) + b2_ref[...]
    o_ref[...] = h_ref[...] + _ln(y, g_ref[...], bl_ref[...])


def _node_mlp(h, partials, p, tile):
    N, H = h.shape
    assert N % tile == 0
    nb = N // tile
    row = pl.BlockSpec((tile, H), lambda i: (i, 0))
    return pl.pallas_call(
        _node_kernel,
        grid=(nb,),
        in_specs=[row,
                  pl.BlockSpec((tile, H), lambda i: (i, 0)),
                  pl.BlockSpec((tile, H), lambda i: (i + nb, 0)),
                  _full_spec((2 * H, H)), _full_spec((1, H)),
                  _full_spec((H, H)), _full_spec((1, H)),
                  _full_spec((1, H)), _full_spec((1, H))],
        out_specs=row,
        out_shape=jax.ShapeDtypeStruct((N, H), jnp.float32),
    )(h, partials, partials, p["w1"], p["b1"].reshape(1, -1), p["w2"],
      p["b2"].reshape(1, -1), p["g"].reshape(1, -1), p["bl"].reshape(1, -1))


def _dec_kernel(h_ref, w1_ref, b1_ref, w2_ref, b2_ref, o_ref):
    z = jnp.maximum(_dot(h_ref[...], w1_ref[...]) + b1_ref[...], 0.0)
    o_ref[...] = _dot(z, w2_ref[...]) + b2_ref[...]


def _decode(h, p, tile):
    N, H = h.shape
    out_d = p["w2"].shape[1]
    return pl.pallas_call(
        _dec_kernel,
        grid=(N // tile,),
        in_specs=[pl.BlockSpec((tile, H), lambda i: (i, 0)),
                  _full_spec((H, H)), _full_spec((1, H)),
                  _full_spec((H, out_d)), _full_spec((1, out_d))],
        out_specs=pl.BlockSpec((tile, out_d), lambda i: (i, 0)),
        out_shape=jax.ShapeDtypeStruct((N, out_d), jnp.float32),
    )(h, p["w1"], p["b1"].reshape(1, -1), p["w2"], p["b2"].reshape(1, -1))


def kernel(x, edge_index, edge_attr, params):
    N = x.shape[0]
    E = edge_attr.shape[0]
    H = params["enc_node"]["w2"].shape[1]
    senders = edge_index[0]
    receivers = edge_index[1]

    h = _encode(x, params["enc_node"], tile=2000)
    e = _encode(edge_attr, params["enc_edge"], tile=2560)
    zeros = jnp.zeros((NC * N, H), jnp.float32)

    NCHUNK = 2
    unit = NW * G
    groups = E // unit
    per = groups // NCHUNK
    sizes = [per * unit] * (NCHUNK - 1)
    sizes.append(E - sum(sizes))
    offs = [sum(sizes[:i]) for i in range(NCHUNK)]
    sch = [senders[o:o + n] for o, n in zip(offs, sizes)]
    rch = [receivers[o:o + n] for o, n in zip(offs, sizes)]
    ech = [e[o:o + n] for o, n in zip(offs, sizes)]

    for blk in params["blocks"]:
        w1 = blk["edge"]["w1"]
        ps, pr = _project(h, w1[0:H], w1[H:2 * H], tile=2000)
        gs = [_sc_gather_sum(ps, pr, s, r) for s, r in zip(sch, rch)]
        en = []
        for i in range(NCHUNK):
            e_new, ech[i] = _edge_mlp(gs[i], ech[i], blk["edge"], tile=2560)
            en.append(e_new)
        p = zeros
        for i in range(NCHUNK):
            p = _sc_scatter_add(en[i], rch[i], p)
        h = _node_mlp(h, p, blk["node"], tile=2000)

    return _decode(h, params["dec"], tile=2000)
